# baseline (speedup 1.0000x reference)
import jax
import jax.numpy as jnp
from jax.experimental import pallas as pl

_BH = 32


def _fill_kernel(row_ref, col_ref, depth_ref, out_ref):
    _, bh, w, _ = out_ref.shape
    di = pl.program_id(0)
    hi = pl.program_id(1)
    x = col_ref[:w, :]
    y = row_ref[pl.ds(hi * bh, bh), :]
    z = depth_ref[pl.ds(di, 1), :]
    xb = jnp.broadcast_to(x[None, :, :], (bh, w, 64))
    yb = jnp.broadcast_to(y[:, None, :], (bh, w, 64))
    zb = jnp.broadcast_to(z[:, None, :], (bh, w, 64))
    out_ref[0] = jnp.concatenate([xb, yb, zb], axis=-1)


def kernel(scan, row_weight, col_weight, depth_weight):
    d, em, h, w = scan.shape
    c = row_weight.shape[1] + col_weight.shape[1] + depth_weight.shape[1]
    bh = _BH if h % _BH == 0 else 8
    grid = (d, h // bh)
    return pl.pallas_call(
        _fill_kernel,
        grid=grid,
        in_specs=[
            pl.BlockSpec(row_weight.shape, lambda di, hi: (0, 0)),
            pl.BlockSpec(col_weight.shape, lambda di, hi: (0, 0)),
            pl.BlockSpec(depth_weight.shape, lambda di, hi: (0, 0)),
        ],
        out_specs=pl.BlockSpec((1, bh, w, c), lambda di, hi: (di, hi, 0, 0)),
        out_shape=jax.ShapeDtypeStruct((d, h, w, c), jnp.float32),
    )(row_weight, col_weight, depth_weight)
